# baseline (speedup 1.0000x reference)
import jax
import jax.numpy as jnp
from jax.experimental import pallas as pl
from jax.experimental.pallas import tpu as pltpu


_BM = 256


def _proj_body(seq_ref, wt_ref, fts_ref):
    fts_ref[...] = jnp.dot(
        seq_ref[...], wt_ref[...], preferred_element_type=jnp.float32
    )


def _agg_body(fts_ref, adj_ref, bias_ref, alpha_ref, out_ref):
    o = jnp.dot(adj_ref[...], fts_ref[...], preferred_element_type=jnp.float32)
    o = o + bias_ref[...]
    alpha = alpha_ref[0]
    out_ref[...] = jnp.where(o >= 0, o, alpha * o)


def kernel(seq, adj, W, bias, alpha):
    b, n, d_in = seq.shape
    d_out = W.shape[0]
    seq2 = seq.reshape(n, d_in)
    adj2 = adj.reshape(n, n)
    wt = W.T
    bias2 = bias.reshape(1, d_out)
    alpha2 = jnp.reshape(alpha, (1,))

    fts = pl.pallas_call(
        _proj_body,
        out_shape=jax.ShapeDtypeStruct((n, d_out), jnp.float32),
    )(seq2, wt)

    out = pl.pallas_call(
        _agg_body,
        grid=(pl.cdiv(n, _BM),),
        in_specs=[
            pl.BlockSpec((n, d_out), lambda i: (0, 0)),
            pl.BlockSpec((_BM, n), lambda i: (i, 0)),
            pl.BlockSpec((1, d_out), lambda i: (0, 0)),
            pl.BlockSpec(memory_space=pltpu.SMEM),
        ],
        out_specs=pl.BlockSpec((_BM, d_out), lambda i: (i, 0)),
        out_shape=jax.ShapeDtypeStruct((n, d_out), jnp.float32),
        compiler_params=pltpu.CompilerParams(
            dimension_semantics=("parallel",),
        ),
    )(fts, adj2, bias2, alpha2)
    return out.reshape(b, n, d_out)


# speedup vs baseline: 1.0219x; 1.0219x over previous
import jax
import jax.numpy as jnp
from jax.experimental import pallas as pl
from jax.experimental.pallas import tpu as pltpu


_BM = 256


def _gcn_body(seq_ref, adj_ref, wt_ref, bias_ref, alpha_ref, out_ref, fts_ref):
    @pl.when(pl.program_id(0) == 0)
    def _():
        fts_ref[...] = jnp.dot(
            seq_ref[...], wt_ref[...], preferred_element_type=jnp.float32
        )

    o = jnp.dot(adj_ref[...], fts_ref[...], preferred_element_type=jnp.float32)
    o = o + bias_ref[...]
    alpha = alpha_ref[0]
    out_ref[...] = jnp.where(o >= 0, o, alpha * o)


def kernel(seq, adj, W, bias, alpha):
    b, n, d_in = seq.shape
    d_out = W.shape[0]
    seq2 = seq.reshape(n, d_in)
    adj2 = adj.reshape(n, n)
    wt = W.T
    bias2 = bias.reshape(1, d_out)
    alpha2 = jnp.reshape(alpha, (1,))

    grid = (pl.cdiv(n, _BM),)
    out = pl.pallas_call(
        _gcn_body,
        grid=grid,
        in_specs=[
            pl.BlockSpec((n, d_in), lambda i: (0, 0)),
            pl.BlockSpec((_BM, n), lambda i: (i, 0)),
            pl.BlockSpec((d_in, d_out), lambda i: (0, 0)),
            pl.BlockSpec((1, d_out), lambda i: (0, 0)),
            pl.BlockSpec(memory_space=pltpu.SMEM),
        ],
        out_specs=pl.BlockSpec((_BM, d_out), lambda i: (i, 0)),
        out_shape=jax.ShapeDtypeStruct((n, d_out), jnp.float32),
        scratch_shapes=[pltpu.VMEM((n, d_out), jnp.float32)],
    )(seq2, adj2, wt, bias2, alpha2)
    return out.reshape(b, n, d_out)


# speedup vs baseline: 1.0244x; 1.0024x over previous
import jax
import jax.numpy as jnp
from jax.experimental import pallas as pl
from jax.experimental.pallas import tpu as pltpu


_CH = 200
_NBUF = 4


def _gcn_body(seq_ref, adj_hbm, wt_ref, bias_ref, alpha_ref, out_hbm,
              fts_ref, ring_ref, ostage_ref, in_sems, out_sems):
    n = adj_hbm.shape[0]
    nch = n // _CH

    def in_copy(chunk, slot):
        return pltpu.make_async_copy(
            adj_hbm.at[pl.ds(chunk * _CH, _CH), :],
            ring_ref.at[slot],
            in_sems.at[slot],
        )

    def out_copy(chunk, slot):
        return pltpu.make_async_copy(
            ostage_ref.at[slot],
            out_hbm.at[pl.ds(chunk * _CH, _CH), :],
            out_sems.at[slot],
        )

    for b in range(min(_NBUF, nch)):
        in_copy(b, b).start()
    fts_ref[...] = jnp.dot(
        seq_ref[...], wt_ref[...], preferred_element_type=jnp.float32
    )

    def group(g, _):
        for b in range(_NBUF):
            i = g * _NBUF + b

            @pl.when(i < nch)
            def _():
                in_copy(i, b).wait()
                o = jnp.dot(
                    ring_ref[b], fts_ref[...],
                    preferred_element_type=jnp.float32,
                )
                o = o + bias_ref[...]
                alpha = alpha_ref[0]

                @pl.when(i >= _NBUF)
                def _():
                    out_copy(i - _NBUF, b).wait()

                ostage_ref[b] = jnp.where(o >= 0, o, alpha * o)
                out_copy(i, b).start()

                @pl.when(i + _NBUF < nch)
                def _():
                    in_copy(i + _NBUF, b).start()
        return 0

    jax.lax.fori_loop(0, pl.cdiv(nch, _NBUF), group, 0, unroll=False)

    for c in range(max(0, nch - _NBUF), nch):
        out_copy(c, c % _NBUF).wait()


def kernel(seq, adj, W, bias, alpha):
    b, n, d_in = seq.shape
    d_out = W.shape[0]
    seq2 = seq.reshape(n, d_in)
    adj2 = adj.reshape(n, n)
    wt = W.T
    bias2 = bias.reshape(1, d_out)
    alpha2 = jnp.reshape(alpha, (1,))

    out = pl.pallas_call(
        _gcn_body,
        in_specs=[
            pl.BlockSpec((n, d_in), lambda: (0, 0)),
            pl.BlockSpec(memory_space=pl.ANY),
            pl.BlockSpec((d_in, d_out), lambda: (0, 0)),
            pl.BlockSpec((1, d_out), lambda: (0, 0)),
            pl.BlockSpec(memory_space=pltpu.SMEM),
        ],
        out_specs=pl.BlockSpec(memory_space=pl.ANY),
        out_shape=jax.ShapeDtypeStruct((n, d_out), jnp.float32),
        scratch_shapes=[
            pltpu.VMEM((n, d_out), jnp.float32),
            pltpu.VMEM((_NBUF, _CH, n), jnp.float32),
            pltpu.VMEM((_NBUF, _CH, d_out), jnp.float32),
            pltpu.SemaphoreType.DMA((_NBUF,)),
            pltpu.SemaphoreType.DMA((_NBUF,)),
        ],
    )(seq2, adj2, wt, bias2, alpha2)
    return out.reshape(b, n, d_out)


# speedup vs baseline: 1.0258x; 1.0014x over previous
import jax
import jax.numpy as jnp
from jax.experimental import pallas as pl
from jax.experimental.pallas import tpu as pltpu


_CH = 80
_NBUF = 8


def _gcn_body(seq_ref, adj_hbm, wt_ref, bias_ref, alpha_ref, out_hbm,
              fts_ref, ring_ref, ostage_ref, in_sems, out_sems):
    n = adj_hbm.shape[0]
    nch = n // _CH

    def in_copy(chunk, slot):
        return pltpu.make_async_copy(
            adj_hbm.at[pl.ds(chunk * _CH, _CH), :],
            ring_ref.at[slot],
            in_sems.at[slot],
        )

    def out_copy(chunk, slot):
        return pltpu.make_async_copy(
            ostage_ref.at[slot],
            out_hbm.at[pl.ds(chunk * _CH, _CH), :],
            out_sems.at[slot],
        )

    for b in range(min(_NBUF, nch)):
        in_copy(b, b).start()
    fts_ref[...] = jnp.dot(
        seq_ref[...], wt_ref[...], preferred_element_type=jnp.float32
    )

    def group(g, _):
        for b in range(_NBUF):
            i = g * _NBUF + b

            @pl.when(i < nch)
            def _():
                in_copy(i, b).wait()
                o = jnp.dot(
                    ring_ref[b], fts_ref[...],
                    preferred_element_type=jnp.float32,
                )
                o = o + bias_ref[...]
                alpha = alpha_ref[0]

                @pl.when(i >= _NBUF)
                def _():
                    out_copy(i - _NBUF, b).wait()

                ostage_ref[b] = jnp.where(o >= 0, o, alpha * o)
                out_copy(i, b).start()

                @pl.when(i + _NBUF < nch)
                def _():
                    in_copy(i + _NBUF, b).start()
        return 0

    jax.lax.fori_loop(0, pl.cdiv(nch, _NBUF), group, 0, unroll=False)

    for c in range(max(0, nch - _NBUF), nch):
        out_copy(c, c % _NBUF).wait()


def kernel(seq, adj, W, bias, alpha):
    b, n, d_in = seq.shape
    d_out = W.shape[0]
    seq2 = seq.reshape(n, d_in)
    adj2 = adj.reshape(n, n)
    wt = W.T
    bias2 = bias.reshape(1, d_out)
    alpha2 = jnp.reshape(alpha, (1,))

    out = pl.pallas_call(
        _gcn_body,
        in_specs=[
            pl.BlockSpec((n, d_in), lambda: (0, 0)),
            pl.BlockSpec(memory_space=pl.ANY),
            pl.BlockSpec((d_in, d_out), lambda: (0, 0)),
            pl.BlockSpec((1, d_out), lambda: (0, 0)),
            pl.BlockSpec(memory_space=pltpu.SMEM),
        ],
        out_specs=pl.BlockSpec(memory_space=pl.ANY),
        out_shape=jax.ShapeDtypeStruct((n, d_out), jnp.float32),
        scratch_shapes=[
            pltpu.VMEM((n, d_out), jnp.float32),
            pltpu.VMEM((_NBUF, _CH, n), jnp.float32),
            pltpu.VMEM((_NBUF, _CH, d_out), jnp.float32),
            pltpu.SemaphoreType.DMA((_NBUF,)),
            pltpu.SemaphoreType.DMA((_NBUF,)),
        ],
    )(seq2, adj2, wt, bias2, alpha2)
    return out.reshape(b, n, d_out)


# speedup vs baseline: 1.0461x; 1.0198x over previous
import jax
import jax.numpy as jnp
from jax.experimental import pallas as pl
from jax.experimental.pallas import tpu as pltpu


_BM = 288


def _gcn_body(seq_ref, adj_ref, wt_ref, bias_ref, alpha_ref, out_ref, fts_ref):
    @pl.when(pl.program_id(0) == 0)
    def _():
        fts_ref[...] = jnp.dot(
            seq_ref[...], wt_ref[...], preferred_element_type=jnp.float32
        )

    o = jnp.dot(adj_ref[...], fts_ref[...], preferred_element_type=jnp.float32)
    o = o + bias_ref[...]
    alpha = alpha_ref[0]
    out_ref[...] = jnp.where(o >= 0, o, alpha * o)


def kernel(seq, adj, W, bias, alpha):
    b, n, d_in = seq.shape
    d_out = W.shape[0]
    seq2 = seq.reshape(n, d_in)
    adj2 = adj.reshape(n, n)
    wt = W.T
    bias2 = bias.reshape(1, d_out)
    alpha2 = jnp.reshape(alpha, (1,))

    grid = (pl.cdiv(n, _BM),)
    out = pl.pallas_call(
        _gcn_body,
        grid=grid,
        in_specs=[
            pl.BlockSpec((n, d_in), lambda i: (0, 0)),
            pl.BlockSpec((_BM, n), lambda i: (i, 0)),
            pl.BlockSpec((d_in, d_out), lambda i: (0, 0)),
            pl.BlockSpec((1, d_out), lambda i: (0, 0)),
            pl.BlockSpec(memory_space=pltpu.SMEM),
        ],
        out_specs=pl.BlockSpec((_BM, d_out), lambda i: (i, 0)),
        out_shape=jax.ShapeDtypeStruct((n, d_out), jnp.float32),
        scratch_shapes=[pltpu.VMEM((n, d_out), jnp.float32)],
    )(seq2, adj2, wt, bias2, alpha2)
    return out.reshape(b, n, d_out)
